# Initial kernel scaffold; baseline (speedup 1.0000x reference)
import jax
import jax.numpy as jnp
from jax.experimental import pallas as pl


def kernel(x, emb, Wq, bq, Aq, Bq, Wk, bk, Ak, Bk, Wr, We, be, Ae, Be, Wv, bv):
    raise NotImplementedError("write your pallas kernel here")


# speedup vs baseline: 1.3082x; 1.3082x over previous
import math

import jax
import jax.numpy as jnp
from jax import lax
from jax.experimental import pallas as pl
from jax.experimental.pallas import tpu as pltpu
from jax.experimental.pallas import tpu_sc as plsc

B, S, H, V, E, R = 1, 2048, 1024, 32000, 8, 4
SCALING = 1.0 / R
RSQRT_H = 1.0 / math.sqrt(H)

S_TILE = 256
N_S = S // S_TILE
V_TILE = 1280
N_V = V // V_TILE

_NC, _NS = 2, 16
_NW = _NC * _NS
_BPW = S // _NW


def _gather_body(table_hbm, idx_hbm, out_hbm, idx_v, rows_v, sem):
    wid = lax.axis_index("s") * _NC + lax.axis_index("c")
    base = wid * _BPW
    pltpu.sync_copy(idx_hbm.at[pl.ds(base, _BPW)], idx_v)
    pltpu.async_copy(table_hbm.at[idx_v], rows_v, sem).wait()
    pltpu.sync_copy(rows_v, out_hbm.at[pl.ds(base, _BPW)])


def _sc_gather(emb, idx):
    mesh = plsc.VectorSubcoreMesh(core_axis_name="c", subcore_axis_name="s")
    return pl.kernel(
        _gather_body,
        mesh=mesh,
        out_type=jax.ShapeDtypeStruct((S, H), jnp.float32),
        scratch_types=[
            pltpu.VMEM((_BPW,), jnp.int32),
            pltpu.VMEM((_BPW, H), jnp.float32),
            pltpu.SemaphoreType.DMA,
        ],
    )(emb, idx)


def _qk_body(h_ref, Wq_ref, bq_ref, Aq_ref, Bq_ref, Wk_ref, bk_ref, Ak_ref,
             Bk_ref, Wr_ref, q_ref, k_ref, w_ref):
    h = h_ref[...]
    q = jnp.dot(h, Wq_ref[...], preferred_element_type=jnp.float32)
    q += jnp.dot(jnp.dot(h, Aq_ref[...], preferred_element_type=jnp.float32),
                 Bq_ref[...], preferred_element_type=jnp.float32) * SCALING
    q_ref[...] = q + bq_ref[...]
    k = jnp.dot(h, Wk_ref[...], preferred_element_type=jnp.float32)
    k += jnp.dot(jnp.dot(h, Ak_ref[...], preferred_element_type=jnp.float32),
                 Bk_ref[...], preferred_element_type=jnp.float32) * SCALING
    k_ref[...] = k + bk_ref[...]
    rl = jnp.dot(h, Wr_ref[...], preferred_element_type=jnp.float32)
    eidx = lax.broadcasted_iota(jnp.int32, (S_TILE, E), 1)
    m1 = jnp.max(rl, axis=-1, keepdims=True)
    i1 = jnp.min(jnp.where(rl == m1, eidx, E), axis=-1, keepdims=True)
    rl2 = jnp.where(eidx == i1, -jnp.inf, rl)
    m2 = jnp.max(rl2, axis=-1, keepdims=True)
    i2 = jnp.min(jnp.where(rl2 == m2, eidx, E), axis=-1, keepdims=True)
    t = jnp.exp(m2 - m1)
    w1 = 1.0 / (1.0 + t)
    w2 = w1 * t
    w_ref[...] = jnp.where(eidx == i1, w1, 0.0) + jnp.where(eidx == i2, w2, 0.0)


def _qk_router(h, Wq, bq, Aq, Bq, Wk, bk, Ak, Bk, Wr):
    return pl.pallas_call(
        _qk_body,
        grid=(N_S,),
        in_specs=[
            pl.BlockSpec((S_TILE, H), lambda i: (i, 0)),
            pl.BlockSpec((H, H), lambda i: (0, 0)),
            pl.BlockSpec((H,), lambda i: (0,)),
            pl.BlockSpec((H, R), lambda i: (0, 0)),
            pl.BlockSpec((R, H), lambda i: (0, 0)),
            pl.BlockSpec((H, H), lambda i: (0, 0)),
            pl.BlockSpec((H,), lambda i: (0,)),
            pl.BlockSpec((H, R), lambda i: (0, 0)),
            pl.BlockSpec((R, H), lambda i: (0, 0)),
            pl.BlockSpec((H, E), lambda i: (0, 0)),
        ],
        out_specs=[
            pl.BlockSpec((S_TILE, H), lambda i: (i, 0)),
            pl.BlockSpec((S_TILE, H), lambda i: (i, 0)),
            pl.BlockSpec((S_TILE, E), lambda i: (i, 0)),
        ],
        out_shape=[
            jax.ShapeDtypeStruct((S, H), jnp.float32),
            jax.ShapeDtypeStruct((S, H), jnp.float32),
            jax.ShapeDtypeStruct((S, E), jnp.float32),
        ],
    )(h, Wq, bq, Aq, Bq, Wk, bk, Ak, Bk, Wr)


def _att_body(q_ref, k_ref, imp_ref):
    att = lax.dot_general(q_ref[...], k_ref[...], (((1,), (1,)), ((), ())),
                          preferred_element_type=jnp.float32) * RSQRT_H
    m = jnp.max(att, axis=-1, keepdims=True)
    p = jnp.exp(att - m)
    probs = p / jnp.sum(p, axis=-1, keepdims=True)
    colsum = jnp.sum(probs, axis=0, keepdims=True)

    @pl.when(pl.program_id(0) == 0)
    def _():
        imp_ref[...] = colsum

    @pl.when(pl.program_id(0) != 0)
    def _():
        imp_ref[...] += colsum


def _attention(q, k):
    return pl.pallas_call(
        _att_body,
        grid=(N_S,),
        in_specs=[
            pl.BlockSpec((S_TILE, H), lambda i: (i, 0)),
            pl.BlockSpec((S, H), lambda i: (0, 0)),
        ],
        out_specs=pl.BlockSpec((1, S), lambda i: (0, 0)),
        out_shape=jax.ShapeDtypeStruct((1, S), jnp.float32),
    )(q, k)


def _moe_body(h_ref, We_ref, be_ref, Ae_ref, Be_ref, w_ref, out_ref):
    e = pl.program_id(1)
    h = h_ref[...]
    mm = jnp.dot(h, We_ref[0], preferred_element_type=jnp.float32)
    mm += jnp.dot(jnp.dot(h, Ae_ref[0], preferred_element_type=jnp.float32),
                  Be_ref[0], preferred_element_type=jnp.float32) * SCALING
    mm += be_ref[0]
    sel = (lax.broadcasted_iota(jnp.int32, (S_TILE, E), 1) == e)
    wcol = jnp.sum(w_ref[...] * sel.astype(jnp.float32), axis=1, keepdims=True)
    contrib = mm * wcol

    @pl.when(e == 0)
    def _():
        out_ref[...] = contrib

    @pl.when(e != 0)
    def _():
        out_ref[...] += contrib


def _moe(h, We, be, Ae, Be, w):
    return pl.pallas_call(
        _moe_body,
        grid=(N_S, E),
        in_specs=[
            pl.BlockSpec((S_TILE, H), lambda s, e: (s, 0)),
            pl.BlockSpec((1, H, H), lambda s, e: (e, 0, 0)),
            pl.BlockSpec((1, 1, H), lambda s, e: (e, 0, 0)),
            pl.BlockSpec((1, H, R), lambda s, e: (e, 0, 0)),
            pl.BlockSpec((1, R, H), lambda s, e: (e, 0, 0)),
            pl.BlockSpec((S_TILE, E), lambda s, e: (s, 0)),
        ],
        out_specs=pl.BlockSpec((S_TILE, H), lambda s, e: (s, 0)),
        out_shape=jax.ShapeDtypeStruct((S, H), jnp.float32),
    )(h, We, be.reshape(E, 1, H), Ae, Be, w)


def _vocab_body(moe_ref, Wv_ref, bv_ref, out_ref):
    out_ref[...] = (jnp.dot(moe_ref[...], Wv_ref[...],
                            preferred_element_type=jnp.float32) + bv_ref[...])


def _vocab(moe, Wv, bv):
    return pl.pallas_call(
        _vocab_body,
        grid=(N_V,),
        in_specs=[
            pl.BlockSpec((S, H), lambda j: (0, 0)),
            pl.BlockSpec((H, V_TILE), lambda j: (0, j)),
            pl.BlockSpec((1, V_TILE), lambda j: (0, j)),
        ],
        out_specs=pl.BlockSpec((S, V_TILE), lambda j: (0, j)),
        out_shape=jax.ShapeDtypeStruct((S, V), jnp.float32),
    )(moe, Wv, bv.reshape(1, V))


def kernel(x, emb, Wq, bq, Aq, Bq, Wk, bk, Ak, Bk, Wr, We, be, Ae, Be, Wv, bv):
    idx = x.reshape(S).astype(jnp.int32)
    h = _sc_gather(emb, idx)
    q, k, w = _qk_router(h, Wq, bq, Aq, Bq, Wk, bk, Ak, Bk, Wr)
    imp = _attention(q, k)
    moe_out = _moe(h, We, be, Ae, Be, w)
    logits = _vocab(moe_out, Wv, bv)
    return (logits.reshape(B, S, V), imp)
